# baseline (speedup 1.0000x reference)
import jax
import jax.numpy as jnp
from jax import lax
from jax.experimental import pallas as pl
from jax.experimental.pallas import tpu as pltpu
from jax.experimental.pallas import tpu_sc as plsc

_B, _T, _D = 4, 8192, 768

_INFO = plsc.get_sparse_core_info()
_NC = _INFO.num_cores
_NS = _INFO.num_subcores
_NW = _NC * _NS
_ROWS = _T // _NW
_CHUNK = 64
_NCHUNK = _ROWS // _CHUNK


def _sc_body(table_hbm, out_hbm, buf0, buf1, sr0, sr1, sw0, sw1):
    wid = lax.axis_index("s") * _NC + lax.axis_index("c")
    base = wid * _ROWS
    bufs = (buf0, buf1)
    srs = (sr0, sr1)
    sws = (sw0, sw1)

    reads = [None, None]
    writes = [None, None]
    reads[0] = pltpu.async_copy(table_hbm.at[pl.ds(base, _CHUNK)], bufs[0], srs[0])
    for ch in range(_NCHUNK):
        i = ch % 2
        ni = (ch + 1) % 2
        if ch + 1 < _NCHUNK:
            if writes[ni] is not None:
                for h in writes[ni]:
                    h.wait()
                writes[ni] = None
            reads[ni] = pltpu.async_copy(
                table_hbm.at[pl.ds(base + (ch + 1) * _CHUNK, _CHUNK)],
                bufs[ni],
                srs[ni],
            )
        reads[i].wait()
        row0 = base + ch * _CHUNK
        writes[i] = [
            pltpu.async_copy(bufs[i], out_hbm.at[b, pl.ds(row0, _CHUNK)], sws[i])
            for b in range(_B)
        ]
    for group in writes:
        if group is not None:
            for h in group:
                h.wait()


def kernel(x, table):
    del x
    mesh = plsc.VectorSubcoreMesh(core_axis_name="c", subcore_axis_name="s")
    run = pl.kernel(
        _sc_body,
        mesh=mesh,
        out_type=jax.ShapeDtypeStruct((_B, _T, _D), jnp.float32),
        scratch_types=[
            pltpu.VMEM((_CHUNK, _D), jnp.float32),
            pltpu.VMEM((_CHUNK, _D), jnp.float32),
            pltpu.SemaphoreType.DMA,
            pltpu.SemaphoreType.DMA,
            pltpu.SemaphoreType.DMA,
            pltpu.SemaphoreType.DMA,
        ],
    )
    return run(table)
